# baseline (speedup 1.0000x reference)
import functools

import jax
import jax.numpy as jnp
import numpy as np
from jax import lax
from jax.experimental import pallas as pl
from jax.experimental.pallas import tpu as pltpu
from jax.experimental.pallas import tpu_sc as plsc

_Z_DIM = 64
_MAX_NORM = float(np.sqrt(_Z_DIM))
_PAD_TRAJ = 80

_NC = 2
_NS = 16
_NW = _NC * _NS


def _make_sc_gather(batch, z_dim):
    rows_per_w = batch // _NW
    mesh = plsc.VectorSubcoreMesh(core_axis_name="c", subcore_axis_name="s")

    @functools.partial(
        pl.kernel,
        mesh=mesh,
        compiler_params=pltpu.CompilerParams(use_tc_tiling_on_sc=False),
        out_type=jax.ShapeDtypeStruct((batch, _PAD_TRAJ, z_dim), jnp.float32),
        scratch_types=[
            pltpu.VMEM((rows_per_w, _PAD_TRAJ), jnp.int32),
            pltpu.VMEM((2, _PAD_TRAJ, z_dim), jnp.float32),
            pltpu.SemaphoreType.DMA,
            pltpu.SemaphoreType.DMA,
        ],
    )
    def sc_gather(noise_hbm, table_hbm, out_hbm, idx_v, rows_v, sem_a, sem_b):
        wid = lax.axis_index("s") * _NC + lax.axis_index("c")
        b0 = wid * rows_per_w
        pltpu.sync_copy(noise_hbm.at[pl.ds(b0, rows_per_w)], idx_v)
        sems = (sem_a, sem_b)
        pltpu.make_async_copy(
            table_hbm.at[idx_v.at[0]], rows_v.at[0], sems[0]).start()
        for j in range(rows_per_w):
            if j + 1 < rows_per_w:
                pltpu.make_async_copy(
                    table_hbm.at[idx_v.at[j + 1]], rows_v.at[(j + 1) % 2],
                    sems[(j + 1) % 2]).start()
            pltpu.make_async_copy(
                table_hbm.at[idx_v.at[j]], rows_v.at[j % 2],
                sems[j % 2]).wait()
            pltpu.sync_copy(rows_v.at[j % 2], out_hbm.at[b0 + j])

    return sc_gather


def _tc_interp_body(nf_ref, te_ref, to_ref, g2_ref, out_ref, *, bb):
    nf_f = nf_ref[0].astype(jnp.float32)
    inv_nf = 1.0 / nf_f
    fh = te_ref.shape[1]
    hp = _PAD_TRAJ // 2
    z = out_ref.shape[-1] // 2

    g2_all = g2_ref[...]
    g2b0 = g2_all.astype(jnp.bfloat16)
    lane_i = lax.broadcasted_iota(jnp.int32, (2 * z, 2 * z), 0)
    col_i = lax.broadcasted_iota(jnp.int32, (2 * z, 2 * z), 1)
    m2 = ((lane_i // z) == (col_i // z)).astype(jnp.bfloat16)
    ss = lax.dot_general(g2b0 * g2b0, m2, (((1,), (0,)), ((), ())),
                         preferred_element_type=jnp.float32)
    scl = jnp.minimum(1.0, _MAX_NORM * lax.rsqrt(jnp.maximum(ss, 1e-24)))
    g2s = (g2_all * scl).astype(jnp.bfloat16)

    j2 = lax.broadcasted_iota(jnp.int32, (hp, fh), 0).astype(jnp.float32)
    je = 2.0 * j2
    jo = je + 1.0
    lane_lo = lax.broadcasted_iota(jnp.int32, (fh, 2 * z), 1) < z

    def hat(jgrid, tf):
        return jnp.maximum(1.0 - jnp.abs(jgrid - tf), 0.0).astype(jnp.bfloat16)

    dims = (((0,), (0,)), ((), ()))
    for b in range(bb):
        g2b = g2s[b * hp:(b + 1) * hp, :]
        g2w = pltpu.roll(g2b, z, 1)
        tfe = te_ref[b:b + 1, :].astype(jnp.float32) * inv_nf
        tfo = to_ref[b:b + 1, :].astype(jnp.float32) * inv_nf
        p = lax.dot_general(hat(je, tfe), g2b, dims,
                            preferred_element_type=jnp.float32)
        q = lax.dot_general(hat(jo, tfe), g2w, dims,
                            preferred_element_type=jnp.float32)
        r = lax.dot_general(hat(je, tfo), g2w, dims,
                            preferred_element_type=jnp.float32)
        s_ = lax.dot_general(hat(jo, tfo), g2b, dims,
                             preferred_element_type=jnp.float32)
        out_ref[b] = jnp.where(lane_lo, p + q, r + s_)


def _make_tc_interp(batch, f_cnt, z_dim, bb):
    hp = _PAD_TRAJ // 2
    fh = f_cnt // 2
    grid = (batch // bb,)
    return pl.pallas_call(
        functools.partial(_tc_interp_body, bb=bb),
        grid=grid,
        in_specs=[
            pl.BlockSpec(memory_space=pltpu.SMEM),
            pl.BlockSpec((bb, fh), lambda b: (b, 0)),
            pl.BlockSpec((bb, fh), lambda b: (b, 0)),
            pl.BlockSpec((bb * hp, 2 * z_dim), lambda b: (b, 0)),
        ],
        out_specs=pl.BlockSpec((bb, fh, 2 * z_dim), lambda b: (b, 0, 0)),
        out_shape=jax.ShapeDtypeStruct((batch, fh, 2 * z_dim), jnp.float32),
    )


def kernel(c, t, l, num_frames_per_motion, motion_noise, embed_weight):
    batch, f_cnt = t.shape
    traj_len = motion_noise.shape[1]
    z_dim = embed_weight.shape[1]
    noise = jnp.pad(motion_noise.astype(jnp.int32),
                    ((0, 0), (0, _PAD_TRAJ - traj_len)))
    gathered = _make_sc_gather(batch, z_dim)(noise, embed_weight)
    g2 = gathered.reshape(batch * (_PAD_TRAJ // 2), 2 * z_dim)
    nf = jnp.asarray(num_frames_per_motion, jnp.int32).reshape(1)
    ti = t.astype(jnp.int32)
    te, to = ti[:, 0::2], ti[:, 1::2]
    bb = 8
    out2 = _make_tc_interp(batch, f_cnt, z_dim, bb)(nf, te, to, g2)
    return out2.reshape(batch, f_cnt, z_dim)


# speedup vs baseline: 2.0664x; 2.0664x over previous
import functools

import jax
import jax.numpy as jnp
import numpy as np
from jax import lax
from jax.experimental import pallas as pl
from jax.experimental.pallas import tpu as pltpu
from jax.experimental.pallas import tpu_sc as plsc

_Z_DIM = 64
_MAX_NORM = float(np.sqrt(_Z_DIM))
_PAD_TRAJ = 80

_NC = 2
_NS = 16
_NW = _NC * _NS


def _make_sc_gather(batch, traj_len, z_dim):
    rows_per_w = batch // _NW
    mesh = plsc.VectorSubcoreMesh(core_axis_name="c", subcore_axis_name="s")

    @functools.partial(
        pl.kernel,
        mesh=mesh,
        compiler_params=pltpu.CompilerParams(use_tc_tiling_on_sc=False),
        out_type=jax.ShapeDtypeStruct((batch, _PAD_TRAJ, z_dim), jnp.float32),
        scratch_types=[
            pltpu.VMEM((rows_per_w, traj_len), jnp.int32),
            pltpu.VMEM((2, _PAD_TRAJ, z_dim), jnp.float32),
            pltpu.SemaphoreType.DMA,
            pltpu.SemaphoreType.DMA,
        ],
    )
    def sc_gather(noise_hbm, table_hbm, out_hbm, idx_v, rows_v, sem_a, sem_b):
        wid = lax.axis_index("s") * _NC + lax.axis_index("c")
        b0 = wid * rows_per_w
        pltpu.sync_copy(noise_hbm.at[pl.ds(b0, rows_per_w)], idx_v)
        zeros16 = jnp.zeros((16,), jnp.float32)
        for k in range(2):
            for r in range(traj_len, _PAD_TRAJ):
                for c in range(z_dim // 16):
                    rows_v[k, r, pl.ds(c * 16, 16)] = zeros16
        sems = (sem_a, sem_b)
        pltpu.make_async_copy(
            table_hbm.at[idx_v.at[0]],
            rows_v.at[0, pl.ds(0, traj_len)], sems[0]).start()
        for j in range(rows_per_w):
            if j + 1 < rows_per_w:
                pltpu.make_async_copy(
                    table_hbm.at[idx_v.at[j + 1]],
                    rows_v.at[(j + 1) % 2, pl.ds(0, traj_len)],
                    sems[(j + 1) % 2]).start()
            pltpu.make_async_copy(
                table_hbm.at[idx_v.at[j]],
                rows_v.at[j % 2, pl.ds(0, traj_len)], sems[j % 2]).wait()
            pltpu.sync_copy(rows_v.at[j % 2], out_hbm.at[b0 + j])

    return sc_gather


def _tc_interp_body(nf_ref, te_ref, to_ref, g2_ref, out_ref, *, bb):
    nf_f = nf_ref[0].astype(jnp.float32)
    inv_nf = 1.0 / nf_f
    fh = te_ref.shape[1]
    hp = _PAD_TRAJ // 2
    z = out_ref.shape[-1] // 2

    g2_all = g2_ref[...]
    g2b0 = g2_all.astype(jnp.bfloat16)
    lane_i = lax.broadcasted_iota(jnp.int32, (2 * z, 2 * z), 0)
    col_i = lax.broadcasted_iota(jnp.int32, (2 * z, 2 * z), 1)
    m2 = ((lane_i // z) == (col_i // z)).astype(jnp.bfloat16)
    ss = lax.dot_general(g2b0 * g2b0, m2, (((1,), (0,)), ((), ())),
                         preferred_element_type=jnp.float32)
    scl = jnp.minimum(1.0, _MAX_NORM * lax.rsqrt(jnp.maximum(ss, 1e-24)))
    g2s = (g2_all * scl).astype(jnp.bfloat16)

    j2 = lax.broadcasted_iota(jnp.int32, (hp, fh), 0).astype(jnp.float32)
    je = 2.0 * j2
    jo = je + 1.0
    lane_lo = lax.broadcasted_iota(jnp.int32, (fh, 2 * z), 1) < z

    def hat(jgrid, tf):
        return jnp.maximum(1.0 - jnp.abs(jgrid - tf), 0.0).astype(jnp.bfloat16)

    dims = (((0,), (0,)), ((), ()))
    for b in range(bb):
        g2b = g2s[b * hp:(b + 1) * hp, :]
        g2w = pltpu.roll(g2b, z, 1)
        tfe = te_ref[b:b + 1, :].astype(jnp.float32) * inv_nf
        tfo = to_ref[b:b + 1, :].astype(jnp.float32) * inv_nf
        p = lax.dot_general(hat(je, tfe), g2b, dims,
                            preferred_element_type=jnp.float32)
        q = lax.dot_general(hat(jo, tfe), g2w, dims,
                            preferred_element_type=jnp.float32)
        r = lax.dot_general(hat(je, tfo), g2w, dims,
                            preferred_element_type=jnp.float32)
        s_ = lax.dot_general(hat(jo, tfo), g2b, dims,
                             preferred_element_type=jnp.float32)
        out_ref[b] = jnp.where(lane_lo, p + q, r + s_)


def _make_tc_interp(batch, f_cnt, z_dim, bb):
    hp = _PAD_TRAJ // 2
    fh = f_cnt // 2
    grid = (batch // bb,)
    return pl.pallas_call(
        functools.partial(_tc_interp_body, bb=bb),
        grid=grid,
        in_specs=[
            pl.BlockSpec(memory_space=pltpu.SMEM),
            pl.BlockSpec((bb, fh), lambda b: (b, 0)),
            pl.BlockSpec((bb, fh), lambda b: (b, 0)),
            pl.BlockSpec((bb * hp, 2 * z_dim), lambda b: (b, 0)),
        ],
        out_specs=pl.BlockSpec((bb, fh, 2 * z_dim), lambda b: (b, 0, 0)),
        out_shape=jax.ShapeDtypeStruct((batch, fh, 2 * z_dim), jnp.float32),
    )


def kernel(c, t, l, num_frames_per_motion, motion_noise, embed_weight):
    batch, f_cnt = t.shape
    traj_len = motion_noise.shape[1]
    z_dim = embed_weight.shape[1]
    noise = motion_noise.astype(jnp.int32)
    gathered = _make_sc_gather(batch, traj_len, z_dim)(noise, embed_weight)
    g2 = gathered.reshape(batch * (_PAD_TRAJ // 2), 2 * z_dim)
    nf = jnp.asarray(num_frames_per_motion, jnp.int32).reshape(1)
    ti = t.astype(jnp.int32)
    te, to = ti[:, 0::2], ti[:, 1::2]
    bb = 8
    out2 = _make_tc_interp(batch, f_cnt, z_dim, bb)(nf, te, to, g2)
    return out2.reshape(batch, f_cnt, z_dim)


# speedup vs baseline: 2.3472x; 1.1359x over previous
import functools

import jax
import jax.numpy as jnp
import numpy as np
from jax import lax
from jax.experimental import pallas as pl
from jax.experimental.pallas import tpu as pltpu
from jax.experimental.pallas import tpu_sc as plsc

_Z_DIM = 64
_MAX_NORM = float(np.sqrt(_Z_DIM))
_PAD_TRAJ = 80

_NC = 2
_NS = 16
_NW = _NC * _NS


def _make_sc_gather(batch, traj_len, z_dim):
    rows_per_w = batch // _NW
    mesh = plsc.VectorSubcoreMesh(core_axis_name="c", subcore_axis_name="s")

    @functools.partial(
        pl.kernel,
        mesh=mesh,
        compiler_params=pltpu.CompilerParams(use_tc_tiling_on_sc=False),
        out_type=jax.ShapeDtypeStruct((batch, _PAD_TRAJ, z_dim), jnp.float32),
        scratch_types=[
            pltpu.VMEM((rows_per_w, traj_len), jnp.int32),
            pltpu.VMEM((2, _PAD_TRAJ, z_dim), jnp.float32),
            pltpu.SemaphoreType.DMA,
            pltpu.SemaphoreType.DMA,
        ],
    )
    def sc_gather(noise_hbm, table_hbm, out_hbm, idx_v, rows_v, sem_a, sem_b):
        wid = lax.axis_index("s") * _NC + lax.axis_index("c")
        b0 = wid * rows_per_w
        pltpu.sync_copy(noise_hbm.at[pl.ds(b0, rows_per_w)], idx_v)
        zeros16 = jnp.zeros((16,), jnp.float32)
        for k in range(2):
            for r in range(traj_len, _PAD_TRAJ):
                for c in range(z_dim // 16):
                    rows_v[k, r, pl.ds(c * 16, 16)] = zeros16
        sems = (sem_a, sem_b)
        pltpu.make_async_copy(
            table_hbm.at[idx_v.at[0]],
            rows_v.at[0, pl.ds(0, traj_len)], sems[0]).start()
        for j in range(rows_per_w):
            if j + 1 < rows_per_w:
                pltpu.make_async_copy(
                    table_hbm.at[idx_v.at[j + 1]],
                    rows_v.at[(j + 1) % 2, pl.ds(0, traj_len)],
                    sems[(j + 1) % 2]).start()
            pltpu.make_async_copy(
                table_hbm.at[idx_v.at[j]],
                rows_v.at[j % 2, pl.ds(0, traj_len)], sems[j % 2]).wait()
            pltpu.sync_copy(rows_v.at[j % 2], out_hbm.at[b0 + j])

    return sc_gather


def _tc_interp_body(nf_ref, te_ref, to_ref, g2_ref, out_ref, *, bb):
    nf_f = nf_ref[0].astype(jnp.float32)
    inv_nf = 1.0 / nf_f
    fh = te_ref.shape[1]
    hp = _PAD_TRAJ // 2
    z = out_ref.shape[-1] // 2

    g2_all = g2_ref[...]
    g2b0 = g2_all.astype(jnp.bfloat16)
    lane_i = lax.broadcasted_iota(jnp.int32, (2 * z, 2 * z), 0)
    col_i = lax.broadcasted_iota(jnp.int32, (2 * z, 2 * z), 1)
    m2 = ((lane_i // z) == (col_i // z)).astype(jnp.bfloat16)
    ss = lax.dot_general(g2b0 * g2b0, m2, (((1,), (0,)), ((), ())),
                         preferred_element_type=jnp.float32)
    scl = jnp.minimum(1.0, _MAX_NORM * lax.rsqrt(jnp.maximum(ss, 1e-24)))
    g2s = (g2_all * scl).astype(jnp.bfloat16)

    j2 = lax.broadcasted_iota(jnp.int32, (hp, fh), 0).astype(jnp.float32)
    je = 2.0 * j2
    jo = je + 1.0
    lane_lo = lax.broadcasted_iota(jnp.int32, (fh, 2 * z), 1) < z

    def hat(jgrid, tf):
        d = (jgrid - tf).astype(jnp.bfloat16)
        one = jnp.bfloat16(1.0)
        zero = jnp.bfloat16(0.0)
        return jnp.maximum(one - jnp.abs(d), zero)

    dims = (((0,), (0,)), ((), ()))
    for b in range(bb):
        g2b = g2s[b * hp:(b + 1) * hp, :]
        g2w = pltpu.roll(g2b, z, 1)
        tfe = te_ref[b:b + 1, :].astype(jnp.float32) * inv_nf
        tfo = to_ref[b:b + 1, :].astype(jnp.float32) * inv_nf
        p = lax.dot_general(hat(je, tfe), g2b, dims,
                            preferred_element_type=jnp.float32)
        q = lax.dot_general(hat(jo, tfe), g2w, dims,
                            preferred_element_type=jnp.float32)
        r = lax.dot_general(hat(je, tfo), g2w, dims,
                            preferred_element_type=jnp.float32)
        s_ = lax.dot_general(hat(jo, tfo), g2b, dims,
                             preferred_element_type=jnp.float32)
        out_ref[b] = jnp.where(lane_lo, p + q, r + s_)


def _make_tc_interp(batch, f_cnt, z_dim, bb):
    hp = _PAD_TRAJ // 2
    fh = f_cnt // 2
    grid = (batch // bb,)
    return pl.pallas_call(
        functools.partial(_tc_interp_body, bb=bb),
        grid=grid,
        in_specs=[
            pl.BlockSpec(memory_space=pltpu.SMEM),
            pl.BlockSpec((bb, fh), lambda b: (b, 0)),
            pl.BlockSpec((bb, fh), lambda b: (b, 0)),
            pl.BlockSpec((bb * hp, 2 * z_dim), lambda b: (b, 0)),
        ],
        out_specs=pl.BlockSpec((bb, fh, 2 * z_dim), lambda b: (b, 0, 0)),
        out_shape=jax.ShapeDtypeStruct((batch, fh, 2 * z_dim), jnp.float32),
    )


def kernel(c, t, l, num_frames_per_motion, motion_noise, embed_weight):
    batch, f_cnt = t.shape
    traj_len = motion_noise.shape[1]
    z_dim = embed_weight.shape[1]
    noise = motion_noise.astype(jnp.int32)
    gathered = _make_sc_gather(batch, traj_len, z_dim)(noise, embed_weight)
    g2 = gathered.reshape(batch * (_PAD_TRAJ // 2), 2 * z_dim)
    nf = jnp.asarray(num_frames_per_motion, jnp.int32).reshape(1)
    ti = t.astype(jnp.int32)
    te, to = ti[:, 0::2], ti[:, 1::2]
    bb = 16
    out2 = _make_tc_interp(batch, f_cnt, z_dim, bb)(nf, te, to, g2)
    return out2.reshape(batch, f_cnt, z_dim)
